# baseline (speedup 1.0000x reference)
import functools
import jax
import jax.numpy as jnp
from jax import lax
from jax.experimental import pallas as pl
from jax.experimental.pallas import tpu as pltpu
from jax.experimental.pallas import tpu_sc as plsc

N = 10000
E = 320000
NC, NS = 2, 16
NW = NC * NS
EW = 10240
EP = NW * EW
CH = 128
NCHUNK = EW // CH
MW = 128
NP = 10240
NSL = NP // NS
GG = 8
NGB = EP // (CH * GG)


@functools.cache
def _mesh():
    return plsc.VectorSubcoreMesh(
        core_axis_name="c", subcore_axis_name="s", num_cores=NC, num_subcores=NS)


def _silu(v):
    return v * (1.0 / (1.0 + jnp.exp(-v)))


K0 = 48
K1 = 160 - K0


def _gather_base(cid, sid):
    kc = K0 + cid * (K1 - K0)
    base = cid * (NS * K0 * CH) + sid * kc * CH
    return base, kc


@functools.cache
def _make_gather1():
    @functools.partial(
        pl.kernel,
        out_type=(jax.ShapeDtypeStruct((EP, 128), jnp.float32),
                  jax.ShapeDtypeStruct((EP, 128), jnp.float32),
                  jax.ShapeDtypeStruct((NGB, GG, 128), jnp.float32),
                  jax.ShapeDtypeStruct((NGB, GG, 128), jnp.float32),
                  jax.ShapeDtypeStruct((NGB, GG, 128), jnp.float32),
                  jax.ShapeDtypeStruct((NGB, GG, 128), jnp.float32)),
        mesh=_mesh(),
        compiler_params=pltpu.CompilerParams(needs_layout_passes=False),
        scratch_types=([pltpu.VMEM((CH,), jnp.int32)] * 4
                       + [pltpu.VMEM((CH, 128), jnp.float32)] * 4
                       + [pltpu.VMEM((4 * N,), jnp.float32),
                          pltpu.VMEM((GG, 128), jnp.float32),
                          pltpu.VMEM((GG, 128), jnp.float32),
                          pltpu.VMEM((GG, 128), jnp.float32),
                          pltpu.VMEM((GG, 128), jnp.float32)]
                       + [pltpu.SemaphoreType.DMA] * 6),
    )
    def gather1_k(td, ts, dsti, srci, pv_h,
                  gd, gs, orpx, orpy, odsq, odvr, *sc):
        idxd = sc[0:2]
        idxs = sc[2:4]
        rowd = sc[4:6]
        rowsv = sc[6:8]
        pv, brpx, brpy, bdsq, bdvr = sc[8:13]
        semi = sc[13:15]
        semg = sc[15:17]
        semw = sc[17:19]
        cid = lax.axis_index("c")
        sid = lax.axis_index("s")
        base, kc = _gather_base(cid, sid)
        pltpu.sync_copy(pv_h, pv)

        for b in range(NBUF):
            offp = pl.multiple_of(base + b * CH, CH)
            pltpu.async_copy(dsti.at[pl.ds(offp, CH)], idxd[b], semi[b])
            pltpu.async_copy(srci.at[pl.ds(offp, CH)], idxs[b], semi[b])

        def body(jo, carry):
            for ji in range(GG):
                b = ji % NBUF
                j = jo * GG + ji
                off = pl.multiple_of(base + j * CH, CH)
                pltpu.make_async_copy(dsti.at[pl.ds(off, CH)], idxd[b], semi[b]).wait()
                pltpu.make_async_copy(srci.at[pl.ds(off, CH)], idxs[b], semi[b]).wait()

                def _drain():
                    pltpu.make_async_copy(rowd[b], gd.at[pl.ds(off, CH)], semw[b]).wait()
                    pltpu.make_async_copy(rowsv[b], gs.at[pl.ds(off, CH)], semw[b]).wait()

                if ji < NBUF:
                    pl.when(jo > 0)(_drain)
                else:
                    _drain()

                cd = pltpu.async_copy(td.at[idxd[b]], rowd[b], semg[b])
                cs = pltpu.async_copy(ts.at[idxs[b]], rowsv[b], semg[b])
                for v in range(CH // 16):
                    sl = pl.ds(v * 16, 16)
                    idv = idxd[b][sl] * 4
                    isv = idxs[b][sl] * 4
                    pxd = plsc.load_gather(pv, [idv])
                    pyd = plsc.load_gather(pv, [idv + 1])
                    vxd = plsc.load_gather(pv, [idv + 2])
                    vyd = plsc.load_gather(pv, [idv + 3])
                    pxs = plsc.load_gather(pv, [isv])
                    pys = plsc.load_gather(pv, [isv + 1])
                    vxs = plsc.load_gather(pv, [isv + 2])
                    vys = plsc.load_gather(pv, [isv + 3])
                    rpx = pxs - pxd
                    rpy = pys - pyd
                    rvx = vxs - vxd
                    rvy = vys - vyd
                    brpx[ji, sl] = rpx
                    brpy[ji, sl] = rpy
                    bdsq[ji, sl] = rpx * rpx + rpy * rpy
                    bdvr[ji, sl] = rvx * rpx + rvy * rpy
                cd.wait()
                cs.wait()

                @pl.when(j + NBUF < kc)
                def _prefetch():
                    offn = pl.multiple_of(off + NBUF * CH, CH)
                    pltpu.async_copy(dsti.at[pl.ds(offn, CH)], idxd[b], semi[b])
                    pltpu.async_copy(srci.at[pl.ds(offn, CH)], idxs[b], semi[b])

                pltpu.async_copy(rowd[b], gd.at[pl.ds(off, CH)], semw[b])
                pltpu.async_copy(rowsv[b], gs.at[pl.ds(off, CH)], semw[b])
            gblk = base // (CH * GG) + jo
            pltpu.sync_copy(brpx, orpx.at[gblk])
            pltpu.sync_copy(brpy, orpy.at[gblk])
            pltpu.sync_copy(bdsq, odsq.at[gblk])
            pltpu.sync_copy(bdvr, odvr.at[gblk])
            return carry

        lax.fori_loop(0, kc // GG, body, 0)
        for b in range(NBUF):
            offp = pl.multiple_of(base + b * CH, CH)
            pltpu.make_async_copy(rowd[b], gd.at[pl.ds(offp, CH)], semw[b]).wait()
            pltpu.make_async_copy(rowsv[b], gs.at[pl.ds(offp, CH)], semw[b]).wait()

    return gather1_k


NBUF = 2


@functools.cache
def _make_gather2():
    @functools.partial(
        pl.kernel,
        out_type=(jax.ShapeDtypeStruct((EP, 128), jnp.float32),
                  jax.ShapeDtypeStruct((EP, 128), jnp.float32)),
        mesh=_mesh(),
        compiler_params=pltpu.CompilerParams(needs_layout_passes=False),
        scratch_types=([pltpu.VMEM((CH,), jnp.int32)] * (2 * NBUF)
                       + [pltpu.VMEM((CH, 128), jnp.float32)] * (2 * NBUF)
                       + [pltpu.SemaphoreType.DMA] * (3 * NBUF)),
    )
    def gather2_k(td, ts, dsti, srci, gd, gs, *sc):
        idxd = sc[0:NBUF]
        idxs = sc[NBUF:2 * NBUF]
        rowd = sc[2 * NBUF:3 * NBUF]
        rowsv = sc[3 * NBUF:4 * NBUF]
        semi = sc[4 * NBUF:5 * NBUF]
        semg = sc[5 * NBUF:6 * NBUF]
        semw = sc[6 * NBUF:7 * NBUF]
        cid = lax.axis_index("c")
        sid = lax.axis_index("s")
        base, kc = _gather_base(cid, sid)

        for b in range(NBUF):
            offp = pl.multiple_of(base + b * CH, CH)
            pltpu.async_copy(dsti.at[pl.ds(offp, CH)], idxd[b], semi[b])
            pltpu.async_copy(srci.at[pl.ds(offp, CH)], idxs[b], semi[b])

        def body(jo, carry):
            for b in range(NBUF):
                j = jo * NBUF + b
                off = pl.multiple_of(base + j * CH, CH)
                pltpu.make_async_copy(dsti.at[pl.ds(off, CH)], idxd[b], semi[b]).wait()
                pltpu.make_async_copy(srci.at[pl.ds(off, CH)], idxs[b], semi[b]).wait()

                @pl.when(jo > 0)
                def _drain():
                    pltpu.make_async_copy(rowd[b], gd.at[pl.ds(off, CH)], semw[b]).wait()
                    pltpu.make_async_copy(rowsv[b], gs.at[pl.ds(off, CH)], semw[b]).wait()

                cd = pltpu.async_copy(td.at[idxd[b]], rowd[b], semg[b])
                cs = pltpu.async_copy(ts.at[idxs[b]], rowsv[b], semg[b])
                cd.wait()
                cs.wait()

                @pl.when(j + NBUF < kc)
                def _prefetch():
                    offn = pl.multiple_of(off + NBUF * CH, CH)
                    pltpu.async_copy(dsti.at[pl.ds(offn, CH)], idxd[b], semi[b])
                    pltpu.async_copy(srci.at[pl.ds(offn, CH)], idxs[b], semi[b])

                pltpu.async_copy(rowd[b], gd.at[pl.ds(off, CH)], semw[b])
                pltpu.async_copy(rowsv[b], gs.at[pl.ds(off, CH)], semw[b])
            return carry

        lax.fori_loop(0, kc // NBUF, body, 0)
        for b in range(NBUF):
            offp = pl.multiple_of(base + b * CH, CH)
            pltpu.make_async_copy(rowd[b], gd.at[pl.ds(offp, CH)], semw[b]).wait()
            pltpu.make_async_copy(rowsv[b], gs.at[pl.ds(offp, CH)], semw[b]).wait()

    return gather2_k


@functools.cache
def _make_scatter():
    @functools.partial(
        pl.kernel,
        out_type=jax.ShapeDtypeStruct((NC * NP, MW), jnp.float32),
        mesh=_mesh(),
        compiler_params=pltpu.CompilerParams(needs_layout_passes=False),
        scratch_types=[pltpu.VMEM((CH,), jnp.int32),
                       pltpu.VMEM((CH, MW), jnp.float32),
                       pltpu.VMEM_SHARED((NP, MW), jnp.float32)],
    )
    def scatter_k(m, dsti, zeros_h, out, idxv, rowv, acc):
        cid = lax.axis_index("c")
        sid = lax.axis_index("s")
        pltpu.sync_copy(zeros_h.at[pl.ds(sid * NSL, NSL)],
                        acc.at[pl.ds(sid * NSL, NSL)])
        plsc.subcore_barrier()
        base = (sid * NC + cid) * EW

        def body(j, carry):
            off = pl.multiple_of(base + j * CH, CH)
            pltpu.sync_copy(dsti.at[pl.ds(off, CH)], idxv)
            pltpu.sync_copy(m.at[pl.ds(off, CH)], rowv)
            pltpu.sync_copy(rowv, acc.at[idxv], add=True)
            return carry

        lax.fori_loop(0, NCHUNK, body, 0)
        plsc.subcore_barrier()
        pltpu.sync_copy(acc.at[pl.ds(sid * NSL, NSL)],
                        out.at[pl.ds(cid * NP + sid * NSL, NSL)])

    return scatter_k


def _gather1_call(td, ts, dsti, srci, pv_h):
    return _make_gather1()(td, ts, dsti, srci, pv_h)


def _gather2_call(td, ts, dsti, srci):
    return _make_gather2()(td, ts, dsti, srci)


def _scatter_call(m, dsti, zeros_h):
    out = _make_scatter()(m, dsti, zeros_h)
    return jnp.reshape(out, (NC, NP, MW))[:, :N, :]


BLK_N = 1000
BLK_E = 1024
GB = BLK_E // CH


def _node_proj(x, wd, ws, cin):
    def body(x_ref, wd_ref, ws_ref, td_ref, ts_ref):
        xb = x_ref[...]
        td_ref[...] = jnp.dot(xb, wd_ref[...], preferred_element_type=jnp.float32)
        ts_ref[...] = jnp.dot(xb, ws_ref[...], preferred_element_type=jnp.float32)

    return pl.pallas_call(
        body,
        grid=(N // BLK_N,),
        in_specs=[pl.BlockSpec((BLK_N, cin), lambda i: (i, 0)),
                  pl.BlockSpec((cin, 128), lambda i: (0, 0)),
                  pl.BlockSpec((cin, 128), lambda i: (0, 0))],
        out_specs=[pl.BlockSpec((BLK_N, 128), lambda i: (i, 0)),
                   pl.BlockSpec((BLK_N, 128), lambda i: (i, 0))],
        out_shape=[jax.ShapeDtypeStruct((N, 128), jnp.float32)] * 2,
    )(x, wd, ws)


def _edge_core(gdb, gsb, rp, dsq, dvr, w3_ref, w2_ref, w3m_ref, bx_ref, bv2_ref):
    pre = (gdb + gsb + w3_ref[0:1, :]
           + dsq * w3_ref[1:2, :] + dvr * w3_ref[2:3, :])
    h1e = _silu(pre[:, 0:64])
    h1v = _silu(pre[:, 64:128])
    h2 = _silu(jnp.dot(h1e, w2_ref[...], preferred_element_type=jnp.float32)
               + bx_ref[0:1, :])
    mh = jnp.dot(h2, w3m_ref[...], preferred_element_type=jnp.float32) + bx_ref[1:2, :]
    vw = jnp.sum(h1v * bx_ref[2:3, :], axis=1, keepdims=True) + bv2_ref[0:1, 0:1]
    mv = vw * rp
    rowid = (jax.lax.broadcasted_iota(jnp.int32, (BLK_E, 1), 0)
             + pl.program_id(0) * BLK_E)
    mrow = jnp.concatenate(
        [mh, mv, rp, dsq, dvr, jnp.zeros((BLK_E, MW - 70), jnp.float32)], axis=1)
    return jnp.where(rowid < E, mrow, 0.0)


def _edge_mlp1(gd, gs, rpx, rpy, dsq, dvr, w3, w2, w3m, bx, bv2):
    def body(gd_ref, gs_ref, rpx_ref, rpy_ref, dsq_ref, dvr_ref,
             w3_ref, w2_ref, w3m_ref, bx_ref, bv2_ref, m_ref):
        r = jax.lax.broadcasted_iota(jnp.int32, (BLK_E, 1), 0)
        sub = jax.lax.broadcasted_iota(jnp.int32, (BLK_E, GG), 1)
        iexp = jnp.where(sub == (r >> 7), 1.0, 0.0)
        lane = jax.lax.broadcasted_iota(jnp.int32, (BLK_E, 128), 1)
        msel = jnp.where(lane == (r & 127), 1.0, 0.0)

        def expand(p_ref):
            c = jnp.dot(iexp, p_ref[0], preferred_element_type=jnp.float32)
            return jnp.sum(c * msel, axis=1, keepdims=True)

        rp = jnp.concatenate([expand(rpx_ref), expand(rpy_ref)], axis=1)
        dsqc = expand(dsq_ref)
        dvrc = expand(dvr_ref)
        m_ref[...] = _edge_core(gd_ref[...], gs_ref[...], rp, dsqc, dvrc,
                                w3_ref, w2_ref, w3m_ref, bx_ref, bv2_ref)

    return pl.pallas_call(
        body,
        grid=(EP // BLK_E,),
        in_specs=[pl.BlockSpec((BLK_E, 128), lambda i: (i, 0)),
                  pl.BlockSpec((BLK_E, 128), lambda i: (i, 0)),
                  pl.BlockSpec((1, GG, 128), lambda i: (i, 0, 0)),
                  pl.BlockSpec((1, GG, 128), lambda i: (i, 0, 0)),
                  pl.BlockSpec((1, GG, 128), lambda i: (i, 0, 0)),
                  pl.BlockSpec((1, GG, 128), lambda i: (i, 0, 0)),
                  pl.BlockSpec((3, 128), lambda i: (0, 0)),
                  pl.BlockSpec((64, 64), lambda i: (0, 0)),
                  pl.BlockSpec((64, 64), lambda i: (0, 0)),
                  pl.BlockSpec((3, 64), lambda i: (0, 0)),
                  pl.BlockSpec((1, 1), lambda i: (0, 0))],
        out_specs=pl.BlockSpec((BLK_E, MW), lambda i: (i, 0)),
        out_shape=jax.ShapeDtypeStruct((EP, MW), jnp.float32),
    )(gd, gs, rpx, rpy, dsq, dvr, w3, w2, w3m, bx, bv2)


def _edge_mlp2(gd, gs, m1, w3, w2, w3m, bx, bv2):
    def body(gd_ref, gs_ref, m1_ref, w3_ref, w2_ref, w3m_ref, bx_ref, bv2_ref,
             m_ref):
        geo = m1_ref[:, 64:80]
        rp = geo[:, 2:4]
        dsq = geo[:, 4:5]
        dvr = geo[:, 5:6]
        m_ref[...] = _edge_core(gd_ref[...], gs_ref[...], rp, dsq, dvr,
                                w3_ref, w2_ref, w3m_ref, bx_ref, bv2_ref)

    return pl.pallas_call(
        body,
        grid=(EP // BLK_E,),
        in_specs=[pl.BlockSpec((BLK_E, 128), lambda i: (i, 0)),
                  pl.BlockSpec((BLK_E, 128), lambda i: (i, 0)),
                  pl.BlockSpec((BLK_E, MW), lambda i: (i, 0)),
                  pl.BlockSpec((3, 128), lambda i: (0, 0)),
                  pl.BlockSpec((64, 64), lambda i: (0, 0)),
                  pl.BlockSpec((64, 64), lambda i: (0, 0)),
                  pl.BlockSpec((3, 64), lambda i: (0, 0)),
                  pl.BlockSpec((1, 1), lambda i: (0, 0))],
        out_specs=pl.BlockSpec((BLK_E, MW), lambda i: (i, 0)),
        out_shape=jax.ShapeDtypeStruct((EP, MW), jnp.float32),
    )(gd, gs, m1, w3, w2, w3m, bx, bv2)


def _node_mid(parts, x, wa, wb, vec, wh2, w2d, w2s):
    def body(p_ref, x_ref, wa_ref, wb_ref, vec_ref, wh2_ref, w2d_ref, w2s_ref,
             h_ref, t2d_ref, t2s_ref):
        p = p_ref[0] + p_ref[1]
        mh = p[:, 0:64]
        mv = p[:, 64:66]
        nrm = jnp.sqrt(jnp.sum(mv * mv, axis=1, keepdims=True) + 1e-12)
        pre = (jnp.dot(x_ref[...], wa_ref[...], preferred_element_type=jnp.float32)
               + jnp.dot(mh, wb_ref[...], preferred_element_type=jnp.float32)
               + nrm * vec_ref[0:1, :] + vec_ref[1:2, :])
        hh = jnp.dot(_silu(pre), wh2_ref[...], preferred_element_type=jnp.float32) + vec_ref[2:3, :]
        g = jnp.maximum(hh, 0.0)
        mu = jnp.mean(g, axis=1, keepdims=True)
        var = jnp.mean(g * g, axis=1, keepdims=True) - mu * mu
        hb = (g - mu) * jax.lax.rsqrt(var + 1e-5) * vec_ref[3:4, :] + vec_ref[4:5, :]
        h_ref[...] = hb
        t2d_ref[...] = jnp.dot(hb, w2d_ref[...], preferred_element_type=jnp.float32)
        t2s_ref[...] = jnp.dot(hb, w2s_ref[...], preferred_element_type=jnp.float32)

    return pl.pallas_call(
        body,
        grid=(N // BLK_N,),
        in_specs=[pl.BlockSpec((2, BLK_N, MW), lambda i: (0, i, 0)),
                  pl.BlockSpec((BLK_N, 128), lambda i: (i, 0)),
                  pl.BlockSpec((128, 64), lambda i: (0, 0)),
                  pl.BlockSpec((64, 64), lambda i: (0, 0)),
                  pl.BlockSpec((5, 64), lambda i: (0, 0)),
                  pl.BlockSpec((64, 64), lambda i: (0, 0)),
                  pl.BlockSpec((64, 128), lambda i: (0, 0)),
                  pl.BlockSpec((64, 128), lambda i: (0, 0))],
        out_specs=[pl.BlockSpec((BLK_N, 64), lambda i: (i, 0)),
                   pl.BlockSpec((BLK_N, 128), lambda i: (i, 0)),
                   pl.BlockSpec((BLK_N, 128), lambda i: (i, 0))],
        out_shape=[jax.ShapeDtypeStruct((N, 64), jnp.float32),
                   jax.ShapeDtypeStruct((N, 128), jnp.float32),
                   jax.ShapeDtypeStruct((N, 128), jnp.float32)],
    )(parts, x, wa, wb, vec, wh2, w2d, w2s)


def _final(parts, x, h, wa, wb, vec, wh2, poolw, poolb, o0, o0b, o1, o1b):
    nblk = N // BLK_N

    def body(p_ref, x_ref, h_ref, wa_ref, wb_ref, vec_ref, wh2_ref,
             poolw_ref, poolb_ref, o0_ref, o0b_ref, o1_ref, o1b_ref,
             s_ref, lat_ref, mu_ref, loss_ref,
             acc_cs, acc_sth, acc_stp, acc_ent):
        i = pl.program_id(0)

        @pl.when(i == 0)
        def _init():
            acc_cs[...] = jnp.zeros_like(acc_cs)
            acc_sth[...] = jnp.zeros_like(acc_sth)
            acc_stp[...] = jnp.zeros_like(acc_stp)
            acc_ent[...] = jnp.zeros_like(acc_ent)

        p = p_ref[0] + p_ref[1]
        mh = p[:, 0:64]
        mv = p[:, 64:66]
        nrm = jnp.sqrt(jnp.sum(mv * mv, axis=1, keepdims=True) + 1e-12)
        pre = (jnp.dot(h_ref[...], wa_ref[...], preferred_element_type=jnp.float32)
               + jnp.dot(mh, wb_ref[...], preferred_element_type=jnp.float32)
               + nrm * vec_ref[0:1, :] + vec_ref[1:2, :])
        hh = jnp.dot(_silu(pre), wh2_ref[...], preferred_element_type=jnp.float32) + vec_ref[2:3, :]
        g = jnp.maximum(hh, 0.0)
        mu = jnp.mean(g, axis=1, keepdims=True)
        var = jnp.mean(g * g, axis=1, keepdims=True) - mu * mu
        h2 = (g - mu) * jax.lax.rsqrt(var + 1e-5) * vec_ref[3:4, :] + vec_ref[4:5, :]

        logits = jnp.dot(h2, poolw_ref[...], preferred_element_type=jnp.float32) + poolb_ref[...]
        mx = jnp.max(logits, axis=1, keepdims=True)
        ex = jnp.exp(logits - mx)
        sm = ex / jnp.sum(ex, axis=1, keepdims=True)
        s_ref[...] = sm

        dn = (((0,), (0,)), ((), ()))
        ones = jnp.ones((BLK_N, 1), jnp.float32)
        acc_cs[...] += jax.lax.dot_general(sm, ones, dn, preferred_element_type=jnp.float32)
        acc_sth[...] += jax.lax.dot_general(sm, h2, dn, preferred_element_type=jnp.float32)
        acc_stp[...] += jax.lax.dot_general(sm, x_ref[:, 0:2], dn, preferred_element_type=jnp.float32)
        acc_ent[...] += jnp.reshape(jnp.sum(sm * jnp.log(sm + 1e-8)), (1, 1))

        @pl.when(i == nblk - 1)
        def _fin():
            cs = acc_cs[...]
            denom = cs + 1e-8
            pooled = acc_sth[...] / denom
            mu_ref[...] = acc_stp[...] / denom
            lat = jnp.dot(
                jnp.maximum(jnp.dot(pooled, o0_ref[...], preferred_element_type=jnp.float32)
                            + o0b_ref[...], 0.0),
                o1_ref[...], preferred_element_type=jnp.float32) + o1b_ref[...]
            lat_ref[...] = lat
            mean_s = cs * (1.0 / N)
            load = jnp.sum((mean_s - 1.0 / 32.0) ** 2)
            loss_ref[...] = jnp.concatenate(
                [acc_ent[...] * (-1.0 / N), jnp.reshape(load, (1, 1))], axis=1)

    return pl.pallas_call(
        body,
        grid=(nblk,),
        in_specs=[pl.BlockSpec((2, BLK_N, MW), lambda i: (0, i, 0)),
                  pl.BlockSpec((BLK_N, 128), lambda i: (i, 0)),
                  pl.BlockSpec((BLK_N, 64), lambda i: (i, 0)),
                  pl.BlockSpec((64, 64), lambda i: (0, 0)),
                  pl.BlockSpec((64, 64), lambda i: (0, 0)),
                  pl.BlockSpec((5, 64), lambda i: (0, 0)),
                  pl.BlockSpec((64, 64), lambda i: (0, 0)),
                  pl.BlockSpec((64, 32), lambda i: (0, 0)),
                  pl.BlockSpec((1, 32), lambda i: (0, 0)),
                  pl.BlockSpec((64, 64), lambda i: (0, 0)),
                  pl.BlockSpec((1, 64), lambda i: (0, 0)),
                  pl.BlockSpec((64, 32), lambda i: (0, 0)),
                  pl.BlockSpec((1, 32), lambda i: (0, 0))],
        out_specs=[pl.BlockSpec((BLK_N, 32), lambda i: (i, 0)),
                   pl.BlockSpec((32, 32), lambda i: (0, 0)),
                   pl.BlockSpec((32, 2), lambda i: (0, 0)),
                   pl.BlockSpec((1, 2), lambda i: (0, 0))],
        out_shape=[jax.ShapeDtypeStruct((N, 32), jnp.float32),
                   jax.ShapeDtypeStruct((32, 32), jnp.float32),
                   jax.ShapeDtypeStruct((32, 2), jnp.float32),
                   jax.ShapeDtypeStruct((1, 2), jnp.float32)],
        scratch_shapes=[pltpu.VMEM((32, 1), jnp.float32),
                        pltpu.VMEM((32, 64), jnp.float32),
                        pltpu.VMEM((32, 2), jnp.float32),
                        pltpu.VMEM((1, 1), jnp.float32)],
    )(parts, x, h, wa, wb, vec, wh2, poolw, poolb, o0, o0b, o1, o1b)


def _layer_weights(g, in_ch):
    we0, wv0 = g['phi_e'][0]['w'], g['phi_v'][0]['w']
    wd = jnp.concatenate([we0[:in_ch], wv0[:in_ch]], axis=1)
    ws = jnp.concatenate([we0[in_ch:2 * in_ch], wv0[in_ch:2 * in_ch]], axis=1)
    b1 = jnp.concatenate([g['phi_e'][0]['b'], g['phi_v'][0]['b']])
    wdsq = jnp.concatenate([we0[2 * in_ch], wv0[2 * in_ch]])
    wdvr = jnp.concatenate([we0[2 * in_ch + 1], wv0[2 * in_ch + 1]])
    w3 = jnp.stack([b1, wdsq, wdvr])
    w2 = g['phi_e'][1]['w']
    w3m = g['phi_e'][2]['w']
    bx = jnp.stack([g['phi_e'][1]['b'], g['phi_e'][2]['b'],
                    g['phi_v'][1]['w'][:, 0]])
    bv2 = jnp.reshape(g['phi_v'][1]['b'], (1, 1))
    return wd, ws, w3, w2, w3m, bx, bv2


def _phi_h_weights(g, ln, in_ch):
    ph = g['phi_h']
    wa = ph[0]['w'][:in_ch]
    wb = ph[0]['w'][in_ch:in_ch + 64]
    wn = ph[0]['w'][in_ch + 64]
    vec = jnp.stack([wn, ph[0]['b'], ph[1]['b'], ln['scale'], ln['bias']])
    wh2 = ph[1]['w']
    return wa, wb, vec, wh2


def kernel(x, edge_index, batch, params):
    src = edge_index[0]
    dst = edge_index[1]
    pad = jnp.zeros((EP - E,), jnp.int32)
    dst_p = jnp.concatenate([dst, pad])
    src_p = jnp.concatenate([src, pad])
    zeros_h = jnp.zeros((NP, MW), jnp.float32)
    pv_h = jnp.reshape(x[:, 0:4], (4 * N,))

    g1 = params['g1']
    g2 = params['g2']
    wd1, ws1, w3_1, w2_1, w3m_1, bx_1, bv2_1 = _layer_weights(g1, 128)
    wd2, ws2, w3_2, w2_2, w3m_2, bx_2, bv2_2 = _layer_weights(g2, 64)
    wa1, wb1, vec1, wh21 = _phi_h_weights(g1, params['ln1'], 128)
    wa2, wb2, vec2, wh22 = _phi_h_weights(g2, params['ln2'], 64)

    td, ts = _node_proj(x, wd1, ws1, 128)
    gd, gs, rpx, rpy, dsq, dvr = _gather1_call(td, ts, dst_p, src_p, pv_h)
    m1 = _edge_mlp1(gd, gs, rpx, rpy, dsq, dvr, w3_1, w2_1, w3m_1, bx_1, bv2_1)
    parts1 = _scatter_call(m1, dst_p, zeros_h)

    h, t2d, t2s = _node_mid(parts1, x, wa1, wb1, vec1, wh21, wd2, ws2)

    gd2, gs2 = _gather2_call(t2d, t2s, dst_p, src_p)
    m2 = _edge_mlp2(gd2, gs2, m1, w3_2, w2_2, w3m_2, bx_2, bv2_2)
    parts2 = _scatter_call(m2, dst_p, zeros_h)

    pool = params['pool']
    o0, o1 = params['out'][0], params['out'][1]
    s, latent, mu_c, losses = _final(
        parts2, x, h, wa2, wb2, vec2, wh22,
        pool['w'], jnp.reshape(pool['b'], (1, 32)),
        o0['w'], jnp.reshape(o0['b'], (1, 64)),
        o1['w'], jnp.reshape(o1['b'], (1, 32)))

    return latent, s, losses[0], mu_c


# speedup vs baseline: 1.0259x; 1.0259x over previous
import functools
import jax
import jax.numpy as jnp
from jax import lax
from jax.experimental import pallas as pl
from jax.experimental.pallas import tpu as pltpu
from jax.experimental.pallas import tpu_sc as plsc

N = 10000
E = 320000
NC, NS = 2, 16
NW = NC * NS
EW = 10240
EP = NW * EW
CH = 128
NCHUNK = EW // CH
MW = 128
NP = 10240
NSL = NP // NS
GG = 8
NGB = EP // (CH * GG)


@functools.cache
def _mesh():
    return plsc.VectorSubcoreMesh(
        core_axis_name="c", subcore_axis_name="s", num_cores=NC, num_subcores=NS)


def _silu(v):
    return v * (1.0 / (1.0 + jnp.exp(-v)))


K0 = 112
K1 = 160 - K0


def _gather_base(cid, sid):
    kc = K0 + cid * (K1 - K0)
    base = cid * (NS * K0 * CH) + sid * kc * CH
    return base, kc


@functools.cache
def _make_gather1():
    @functools.partial(
        pl.kernel,
        out_type=(jax.ShapeDtypeStruct((EP, 128), jnp.float32),
                  jax.ShapeDtypeStruct((EP, 128), jnp.float32),
                  jax.ShapeDtypeStruct((NGB, GG, 128), jnp.float32),
                  jax.ShapeDtypeStruct((NGB, GG, 128), jnp.float32),
                  jax.ShapeDtypeStruct((NGB, GG, 128), jnp.float32),
                  jax.ShapeDtypeStruct((NGB, GG, 128), jnp.float32)),
        mesh=_mesh(),
        compiler_params=pltpu.CompilerParams(needs_layout_passes=False),
        scratch_types=([pltpu.VMEM((CH,), jnp.int32)] * 4
                       + [pltpu.VMEM((CH, 128), jnp.float32)] * 4
                       + [pltpu.VMEM((4 * N,), jnp.float32),
                          pltpu.VMEM((GG, 128), jnp.float32),
                          pltpu.VMEM((GG, 128), jnp.float32),
                          pltpu.VMEM((GG, 128), jnp.float32),
                          pltpu.VMEM((GG, 128), jnp.float32)]
                       + [pltpu.SemaphoreType.DMA] * 6),
    )
    def gather1_k(td, ts, dsti, srci, pv_h,
                  gd, gs, orpx, orpy, odsq, odvr, *sc):
        idxd = sc[0:2]
        idxs = sc[2:4]
        rowd = sc[4:6]
        rowsv = sc[6:8]
        pv, brpx, brpy, bdsq, bdvr = sc[8:13]
        semi = sc[13:15]
        semg = sc[15:17]
        semw = sc[17:19]
        cid = lax.axis_index("c")
        sid = lax.axis_index("s")
        base, kc = _gather_base(cid, sid)
        pltpu.sync_copy(pv_h, pv)

        for b in range(NBUF):
            offp = pl.multiple_of(base + b * CH, CH)
            pltpu.async_copy(dsti.at[pl.ds(offp, CH)], idxd[b], semi[b])
            pltpu.async_copy(srci.at[pl.ds(offp, CH)], idxs[b], semi[b])

        def body(jo, carry):
            for ji in range(GG):
                b = ji % NBUF
                j = jo * GG + ji
                off = pl.multiple_of(base + j * CH, CH)
                pltpu.make_async_copy(dsti.at[pl.ds(off, CH)], idxd[b], semi[b]).wait()
                pltpu.make_async_copy(srci.at[pl.ds(off, CH)], idxs[b], semi[b]).wait()

                def _drain():
                    pltpu.make_async_copy(rowd[b], gd.at[pl.ds(off, CH)], semw[b]).wait()
                    pltpu.make_async_copy(rowsv[b], gs.at[pl.ds(off, CH)], semw[b]).wait()

                if ji < NBUF:
                    pl.when(jo > 0)(_drain)
                else:
                    _drain()

                cd = pltpu.async_copy(td.at[idxd[b]], rowd[b], semg[b])
                cs = pltpu.async_copy(ts.at[idxs[b]], rowsv[b], semg[b])
                for v in range(CH // 16):
                    sl = pl.ds(v * 16, 16)
                    idv = idxd[b][sl] * 4
                    isv = idxs[b][sl] * 4
                    pxd = plsc.load_gather(pv, [idv])
                    pyd = plsc.load_gather(pv, [idv + 1])
                    vxd = plsc.load_gather(pv, [idv + 2])
                    vyd = plsc.load_gather(pv, [idv + 3])
                    pxs = plsc.load_gather(pv, [isv])
                    pys = plsc.load_gather(pv, [isv + 1])
                    vxs = plsc.load_gather(pv, [isv + 2])
                    vys = plsc.load_gather(pv, [isv + 3])
                    rpx = pxs - pxd
                    rpy = pys - pyd
                    rvx = vxs - vxd
                    rvy = vys - vyd
                    brpx[ji, sl] = rpx
                    brpy[ji, sl] = rpy
                    bdsq[ji, sl] = rpx * rpx + rpy * rpy
                    bdvr[ji, sl] = rvx * rpx + rvy * rpy
                cd.wait()
                cs.wait()

                @pl.when(j + NBUF < kc)
                def _prefetch():
                    offn = pl.multiple_of(off + NBUF * CH, CH)
                    pltpu.async_copy(dsti.at[pl.ds(offn, CH)], idxd[b], semi[b])
                    pltpu.async_copy(srci.at[pl.ds(offn, CH)], idxs[b], semi[b])

                pltpu.async_copy(rowd[b], gd.at[pl.ds(off, CH)], semw[b])
                pltpu.async_copy(rowsv[b], gs.at[pl.ds(off, CH)], semw[b])
            gblk = base // (CH * GG) + jo
            pltpu.sync_copy(brpx, orpx.at[gblk])
            pltpu.sync_copy(brpy, orpy.at[gblk])
            pltpu.sync_copy(bdsq, odsq.at[gblk])
            pltpu.sync_copy(bdvr, odvr.at[gblk])
            return carry

        lax.fori_loop(0, kc // GG, body, 0)
        for b in range(NBUF):
            offp = pl.multiple_of(base + b * CH, CH)
            pltpu.make_async_copy(rowd[b], gd.at[pl.ds(offp, CH)], semw[b]).wait()
            pltpu.make_async_copy(rowsv[b], gs.at[pl.ds(offp, CH)], semw[b]).wait()

    return gather1_k


NBUF = 2


@functools.cache
def _make_gather2():
    @functools.partial(
        pl.kernel,
        out_type=(jax.ShapeDtypeStruct((EP, 128), jnp.float32),
                  jax.ShapeDtypeStruct((EP, 128), jnp.float32)),
        mesh=_mesh(),
        compiler_params=pltpu.CompilerParams(needs_layout_passes=False),
        scratch_types=([pltpu.VMEM((CH,), jnp.int32)] * (2 * NBUF)
                       + [pltpu.VMEM((CH, 128), jnp.float32)] * (2 * NBUF)
                       + [pltpu.SemaphoreType.DMA] * (3 * NBUF)),
    )
    def gather2_k(td, ts, dsti, srci, gd, gs, *sc):
        idxd = sc[0:NBUF]
        idxs = sc[NBUF:2 * NBUF]
        rowd = sc[2 * NBUF:3 * NBUF]
        rowsv = sc[3 * NBUF:4 * NBUF]
        semi = sc[4 * NBUF:5 * NBUF]
        semg = sc[5 * NBUF:6 * NBUF]
        semw = sc[6 * NBUF:7 * NBUF]
        cid = lax.axis_index("c")
        sid = lax.axis_index("s")
        base, kc = _gather_base(cid, sid)

        for b in range(NBUF):
            offp = pl.multiple_of(base + b * CH, CH)
            pltpu.async_copy(dsti.at[pl.ds(offp, CH)], idxd[b], semi[b])
            pltpu.async_copy(srci.at[pl.ds(offp, CH)], idxs[b], semi[b])

        def body(jo, carry):
            for b in range(NBUF):
                j = jo * NBUF + b
                off = pl.multiple_of(base + j * CH, CH)
                pltpu.make_async_copy(dsti.at[pl.ds(off, CH)], idxd[b], semi[b]).wait()
                pltpu.make_async_copy(srci.at[pl.ds(off, CH)], idxs[b], semi[b]).wait()

                @pl.when(jo > 0)
                def _drain():
                    pltpu.make_async_copy(rowd[b], gd.at[pl.ds(off, CH)], semw[b]).wait()
                    pltpu.make_async_copy(rowsv[b], gs.at[pl.ds(off, CH)], semw[b]).wait()

                cd = pltpu.async_copy(td.at[idxd[b]], rowd[b], semg[b])
                cs = pltpu.async_copy(ts.at[idxs[b]], rowsv[b], semg[b])
                cd.wait()
                cs.wait()

                @pl.when(j + NBUF < kc)
                def _prefetch():
                    offn = pl.multiple_of(off + NBUF * CH, CH)
                    pltpu.async_copy(dsti.at[pl.ds(offn, CH)], idxd[b], semi[b])
                    pltpu.async_copy(srci.at[pl.ds(offn, CH)], idxs[b], semi[b])

                pltpu.async_copy(rowd[b], gd.at[pl.ds(off, CH)], semw[b])
                pltpu.async_copy(rowsv[b], gs.at[pl.ds(off, CH)], semw[b])
            return carry

        lax.fori_loop(0, kc // NBUF, body, 0)
        for b in range(NBUF):
            offp = pl.multiple_of(base + b * CH, CH)
            pltpu.make_async_copy(rowd[b], gd.at[pl.ds(offp, CH)], semw[b]).wait()
            pltpu.make_async_copy(rowsv[b], gs.at[pl.ds(offp, CH)], semw[b]).wait()

    return gather2_k


@functools.cache
def _make_scatter():
    @functools.partial(
        pl.kernel,
        out_type=jax.ShapeDtypeStruct((NC * NP, MW), jnp.float32),
        mesh=_mesh(),
        compiler_params=pltpu.CompilerParams(needs_layout_passes=False),
        scratch_types=[pltpu.VMEM((CH,), jnp.int32),
                       pltpu.VMEM((CH, MW), jnp.float32),
                       pltpu.VMEM_SHARED((NP, MW), jnp.float32)],
    )
    def scatter_k(m, dsti, zeros_h, out, idxv, rowv, acc):
        cid = lax.axis_index("c")
        sid = lax.axis_index("s")
        pltpu.sync_copy(zeros_h.at[pl.ds(sid * NSL, NSL)],
                        acc.at[pl.ds(sid * NSL, NSL)])
        plsc.subcore_barrier()
        base = (sid * NC + cid) * EW

        def body(j, carry):
            off = pl.multiple_of(base + j * CH, CH)
            pltpu.sync_copy(dsti.at[pl.ds(off, CH)], idxv)
            pltpu.sync_copy(m.at[pl.ds(off, CH)], rowv)
            pltpu.sync_copy(rowv, acc.at[idxv], add=True)
            return carry

        lax.fori_loop(0, NCHUNK, body, 0)
        plsc.subcore_barrier()
        pltpu.sync_copy(acc.at[pl.ds(sid * NSL, NSL)],
                        out.at[pl.ds(cid * NP + sid * NSL, NSL)])

    return scatter_k


def _gather1_call(td, ts, dsti, srci, pv_h):
    return _make_gather1()(td, ts, dsti, srci, pv_h)


def _gather2_call(td, ts, dsti, srci):
    return _make_gather2()(td, ts, dsti, srci)


def _scatter_call(m, dsti, zeros_h):
    out = _make_scatter()(m, dsti, zeros_h)
    return jnp.reshape(out, (NC, NP, MW))[:, :N, :]


BLK_N = 1000
BLK_E = 1024
GB = BLK_E // CH


def _node_proj(x, wd, ws, cin):
    def body(x_ref, wd_ref, ws_ref, td_ref, ts_ref):
        xb = x_ref[...]
        td_ref[...] = jnp.dot(xb, wd_ref[...], preferred_element_type=jnp.float32)
        ts_ref[...] = jnp.dot(xb, ws_ref[...], preferred_element_type=jnp.float32)

    return pl.pallas_call(
        body,
        grid=(N // BLK_N,),
        in_specs=[pl.BlockSpec((BLK_N, cin), lambda i: (i, 0)),
                  pl.BlockSpec((cin, 128), lambda i: (0, 0)),
                  pl.BlockSpec((cin, 128), lambda i: (0, 0))],
        out_specs=[pl.BlockSpec((BLK_N, 128), lambda i: (i, 0)),
                   pl.BlockSpec((BLK_N, 128), lambda i: (i, 0))],
        out_shape=[jax.ShapeDtypeStruct((N, 128), jnp.float32)] * 2,
    )(x, wd, ws)


def _edge_core(gdb, gsb, rp, dsq, dvr, w3_ref, w2_ref, w3m_ref, bx_ref, bv2_ref):
    pre = (gdb + gsb + w3_ref[0:1, :]
           + dsq * w3_ref[1:2, :] + dvr * w3_ref[2:3, :])
    h1e = _silu(pre[:, 0:64])
    h1v = _silu(pre[:, 64:128])
    h2 = _silu(jnp.dot(h1e, w2_ref[...], preferred_element_type=jnp.float32)
               + bx_ref[0:1, :])
    mh = jnp.dot(h2, w3m_ref[...], preferred_element_type=jnp.float32) + bx_ref[1:2, :]
    vw = jnp.sum(h1v * bx_ref[2:3, :], axis=1, keepdims=True) + bv2_ref[0:1, 0:1]
    mv = vw * rp
    rowid = (jax.lax.broadcasted_iota(jnp.int32, (BLK_E, 1), 0)
             + pl.program_id(0) * BLK_E)
    mrow = jnp.concatenate(
        [mh, mv, rp, dsq, dvr, jnp.zeros((BLK_E, MW - 70), jnp.float32)], axis=1)
    return jnp.where(rowid < E, mrow, 0.0)


def _edge_mlp1(gd, gs, rpx, rpy, dsq, dvr, w3, w2, w3m, bx, bv2):
    def body(gd_ref, gs_ref, rpx_ref, rpy_ref, dsq_ref, dvr_ref,
             w3_ref, w2_ref, w3m_ref, bx_ref, bv2_ref, m_ref):
        r = jax.lax.broadcasted_iota(jnp.int32, (BLK_E, 1), 0)
        sub = jax.lax.broadcasted_iota(jnp.int32, (BLK_E, GG), 1)
        iexp = jnp.where(sub == (r >> 7), 1.0, 0.0)
        lane = jax.lax.broadcasted_iota(jnp.int32, (BLK_E, 128), 1)
        msel = jnp.where(lane == (r & 127), 1.0, 0.0)

        def expand(p_ref):
            c = jnp.dot(iexp, p_ref[0], preferred_element_type=jnp.float32)
            return jnp.sum(c * msel, axis=1, keepdims=True)

        rp = jnp.concatenate([expand(rpx_ref), expand(rpy_ref)], axis=1)
        dsqc = expand(dsq_ref)
        dvrc = expand(dvr_ref)
        m_ref[...] = _edge_core(gd_ref[...], gs_ref[...], rp, dsqc, dvrc,
                                w3_ref, w2_ref, w3m_ref, bx_ref, bv2_ref)

    return pl.pallas_call(
        body,
        grid=(EP // BLK_E,),
        in_specs=[pl.BlockSpec((BLK_E, 128), lambda i: (i, 0)),
                  pl.BlockSpec((BLK_E, 128), lambda i: (i, 0)),
                  pl.BlockSpec((1, GG, 128), lambda i: (i, 0, 0)),
                  pl.BlockSpec((1, GG, 128), lambda i: (i, 0, 0)),
                  pl.BlockSpec((1, GG, 128), lambda i: (i, 0, 0)),
                  pl.BlockSpec((1, GG, 128), lambda i: (i, 0, 0)),
                  pl.BlockSpec((3, 128), lambda i: (0, 0)),
                  pl.BlockSpec((64, 64), lambda i: (0, 0)),
                  pl.BlockSpec((64, 64), lambda i: (0, 0)),
                  pl.BlockSpec((3, 64), lambda i: (0, 0)),
                  pl.BlockSpec((1, 1), lambda i: (0, 0))],
        out_specs=pl.BlockSpec((BLK_E, MW), lambda i: (i, 0)),
        out_shape=jax.ShapeDtypeStruct((EP, MW), jnp.float32),
    )(gd, gs, rpx, rpy, dsq, dvr, w3, w2, w3m, bx, bv2)


def _edge_mlp2(gd, gs, m1, w3, w2, w3m, bx, bv2):
    def body(gd_ref, gs_ref, m1_ref, w3_ref, w2_ref, w3m_ref, bx_ref, bv2_ref,
             m_ref):
        geo = m1_ref[:, 64:80]
        rp = geo[:, 2:4]
        dsq = geo[:, 4:5]
        dvr = geo[:, 5:6]
        m_ref[...] = _edge_core(gd_ref[...], gs_ref[...], rp, dsq, dvr,
                                w3_ref, w2_ref, w3m_ref, bx_ref, bv2_ref)

    return pl.pallas_call(
        body,
        grid=(EP // BLK_E,),
        in_specs=[pl.BlockSpec((BLK_E, 128), lambda i: (i, 0)),
                  pl.BlockSpec((BLK_E, 128), lambda i: (i, 0)),
                  pl.BlockSpec((BLK_E, MW), lambda i: (i, 0)),
                  pl.BlockSpec((3, 128), lambda i: (0, 0)),
                  pl.BlockSpec((64, 64), lambda i: (0, 0)),
                  pl.BlockSpec((64, 64), lambda i: (0, 0)),
                  pl.BlockSpec((3, 64), lambda i: (0, 0)),
                  pl.BlockSpec((1, 1), lambda i: (0, 0))],
        out_specs=pl.BlockSpec((BLK_E, MW), lambda i: (i, 0)),
        out_shape=jax.ShapeDtypeStruct((EP, MW), jnp.float32),
    )(gd, gs, m1, w3, w2, w3m, bx, bv2)


def _node_mid(parts, x, wa, wb, vec, wh2, w2d, w2s):
    def body(p_ref, x_ref, wa_ref, wb_ref, vec_ref, wh2_ref, w2d_ref, w2s_ref,
             h_ref, t2d_ref, t2s_ref):
        p = p_ref[0] + p_ref[1]
        mh = p[:, 0:64]
        mv = p[:, 64:66]
        nrm = jnp.sqrt(jnp.sum(mv * mv, axis=1, keepdims=True) + 1e-12)
        pre = (jnp.dot(x_ref[...], wa_ref[...], preferred_element_type=jnp.float32)
               + jnp.dot(mh, wb_ref[...], preferred_element_type=jnp.float32)
               + nrm * vec_ref[0:1, :] + vec_ref[1:2, :])
        hh = jnp.dot(_silu(pre), wh2_ref[...], preferred_element_type=jnp.float32) + vec_ref[2:3, :]
        g = jnp.maximum(hh, 0.0)
        mu = jnp.mean(g, axis=1, keepdims=True)
        var = jnp.mean(g * g, axis=1, keepdims=True) - mu * mu
        hb = (g - mu) * jax.lax.rsqrt(var + 1e-5) * vec_ref[3:4, :] + vec_ref[4:5, :]
        h_ref[...] = hb
        t2d_ref[...] = jnp.dot(hb, w2d_ref[...], preferred_element_type=jnp.float32)
        t2s_ref[...] = jnp.dot(hb, w2s_ref[...], preferred_element_type=jnp.float32)

    return pl.pallas_call(
        body,
        grid=(N // BLK_N,),
        in_specs=[pl.BlockSpec((2, BLK_N, MW), lambda i: (0, i, 0)),
                  pl.BlockSpec((BLK_N, 128), lambda i: (i, 0)),
                  pl.BlockSpec((128, 64), lambda i: (0, 0)),
                  pl.BlockSpec((64, 64), lambda i: (0, 0)),
                  pl.BlockSpec((5, 64), lambda i: (0, 0)),
                  pl.BlockSpec((64, 64), lambda i: (0, 0)),
                  pl.BlockSpec((64, 128), lambda i: (0, 0)),
                  pl.BlockSpec((64, 128), lambda i: (0, 0))],
        out_specs=[pl.BlockSpec((BLK_N, 64), lambda i: (i, 0)),
                   pl.BlockSpec((BLK_N, 128), lambda i: (i, 0)),
                   pl.BlockSpec((BLK_N, 128), lambda i: (i, 0))],
        out_shape=[jax.ShapeDtypeStruct((N, 64), jnp.float32),
                   jax.ShapeDtypeStruct((N, 128), jnp.float32),
                   jax.ShapeDtypeStruct((N, 128), jnp.float32)],
    )(parts, x, wa, wb, vec, wh2, w2d, w2s)


def _final(parts, x, h, wa, wb, vec, wh2, poolw, poolb, o0, o0b, o1, o1b):
    nblk = N // BLK_N

    def body(p_ref, x_ref, h_ref, wa_ref, wb_ref, vec_ref, wh2_ref,
             poolw_ref, poolb_ref, o0_ref, o0b_ref, o1_ref, o1b_ref,
             s_ref, lat_ref, mu_ref, loss_ref,
             acc_cs, acc_sth, acc_stp, acc_ent):
        i = pl.program_id(0)

        @pl.when(i == 0)
        def _init():
            acc_cs[...] = jnp.zeros_like(acc_cs)
            acc_sth[...] = jnp.zeros_like(acc_sth)
            acc_stp[...] = jnp.zeros_like(acc_stp)
            acc_ent[...] = jnp.zeros_like(acc_ent)

        p = p_ref[0] + p_ref[1]
        mh = p[:, 0:64]
        mv = p[:, 64:66]
        nrm = jnp.sqrt(jnp.sum(mv * mv, axis=1, keepdims=True) + 1e-12)
        pre = (jnp.dot(h_ref[...], wa_ref[...], preferred_element_type=jnp.float32)
               + jnp.dot(mh, wb_ref[...], preferred_element_type=jnp.float32)
               + nrm * vec_ref[0:1, :] + vec_ref[1:2, :])
        hh = jnp.dot(_silu(pre), wh2_ref[...], preferred_element_type=jnp.float32) + vec_ref[2:3, :]
        g = jnp.maximum(hh, 0.0)
        mu = jnp.mean(g, axis=1, keepdims=True)
        var = jnp.mean(g * g, axis=1, keepdims=True) - mu * mu
        h2 = (g - mu) * jax.lax.rsqrt(var + 1e-5) * vec_ref[3:4, :] + vec_ref[4:5, :]

        logits = jnp.dot(h2, poolw_ref[...], preferred_element_type=jnp.float32) + poolb_ref[...]
        mx = jnp.max(logits, axis=1, keepdims=True)
        ex = jnp.exp(logits - mx)
        sm = ex / jnp.sum(ex, axis=1, keepdims=True)
        s_ref[...] = sm

        dn = (((0,), (0,)), ((), ()))
        ones = jnp.ones((BLK_N, 1), jnp.float32)
        acc_cs[...] += jax.lax.dot_general(sm, ones, dn, preferred_element_type=jnp.float32)
        acc_sth[...] += jax.lax.dot_general(sm, h2, dn, preferred_element_type=jnp.float32)
        acc_stp[...] += jax.lax.dot_general(sm, x_ref[:, 0:2], dn, preferred_element_type=jnp.float32)
        acc_ent[...] += jnp.reshape(jnp.sum(sm * jnp.log(sm + 1e-8)), (1, 1))

        @pl.when(i == nblk - 1)
        def _fin():
            cs = acc_cs[...]
            denom = cs + 1e-8
            pooled = acc_sth[...] / denom
            mu_ref[...] = acc_stp[...] / denom
            lat = jnp.dot(
                jnp.maximum(jnp.dot(pooled, o0_ref[...], preferred_element_type=jnp.float32)
                            + o0b_ref[...], 0.0),
                o1_ref[...], preferred_element_type=jnp.float32) + o1b_ref[...]
            lat_ref[...] = lat
            mean_s = cs * (1.0 / N)
            load = jnp.sum((mean_s - 1.0 / 32.0) ** 2)
            loss_ref[...] = jnp.concatenate(
                [acc_ent[...] * (-1.0 / N), jnp.reshape(load, (1, 1))], axis=1)

    return pl.pallas_call(
        body,
        grid=(nblk,),
        in_specs=[pl.BlockSpec((2, BLK_N, MW), lambda i: (0, i, 0)),
                  pl.BlockSpec((BLK_N, 128), lambda i: (i, 0)),
                  pl.BlockSpec((BLK_N, 64), lambda i: (i, 0)),
                  pl.BlockSpec((64, 64), lambda i: (0, 0)),
                  pl.BlockSpec((64, 64), lambda i: (0, 0)),
                  pl.BlockSpec((5, 64), lambda i: (0, 0)),
                  pl.BlockSpec((64, 64), lambda i: (0, 0)),
                  pl.BlockSpec((64, 32), lambda i: (0, 0)),
                  pl.BlockSpec((1, 32), lambda i: (0, 0)),
                  pl.BlockSpec((64, 64), lambda i: (0, 0)),
                  pl.BlockSpec((1, 64), lambda i: (0, 0)),
                  pl.BlockSpec((64, 32), lambda i: (0, 0)),
                  pl.BlockSpec((1, 32), lambda i: (0, 0))],
        out_specs=[pl.BlockSpec((BLK_N, 32), lambda i: (i, 0)),
                   pl.BlockSpec((32, 32), lambda i: (0, 0)),
                   pl.BlockSpec((32, 2), lambda i: (0, 0)),
                   pl.BlockSpec((1, 2), lambda i: (0, 0))],
        out_shape=[jax.ShapeDtypeStruct((N, 32), jnp.float32),
                   jax.ShapeDtypeStruct((32, 32), jnp.float32),
                   jax.ShapeDtypeStruct((32, 2), jnp.float32),
                   jax.ShapeDtypeStruct((1, 2), jnp.float32)],
        scratch_shapes=[pltpu.VMEM((32, 1), jnp.float32),
                        pltpu.VMEM((32, 64), jnp.float32),
                        pltpu.VMEM((32, 2), jnp.float32),
                        pltpu.VMEM((1, 1), jnp.float32)],
    )(parts, x, h, wa, wb, vec, wh2, poolw, poolb, o0, o0b, o1, o1b)


def _layer_weights(g, in_ch):
    we0, wv0 = g['phi_e'][0]['w'], g['phi_v'][0]['w']
    wd = jnp.concatenate([we0[:in_ch], wv0[:in_ch]], axis=1)
    ws = jnp.concatenate([we0[in_ch:2 * in_ch], wv0[in_ch:2 * in_ch]], axis=1)
    b1 = jnp.concatenate([g['phi_e'][0]['b'], g['phi_v'][0]['b']])
    wdsq = jnp.concatenate([we0[2 * in_ch], wv0[2 * in_ch]])
    wdvr = jnp.concatenate([we0[2 * in_ch + 1], wv0[2 * in_ch + 1]])
    w3 = jnp.stack([b1, wdsq, wdvr])
    w2 = g['phi_e'][1]['w']
    w3m = g['phi_e'][2]['w']
    bx = jnp.stack([g['phi_e'][1]['b'], g['phi_e'][2]['b'],
                    g['phi_v'][1]['w'][:, 0]])
    bv2 = jnp.reshape(g['phi_v'][1]['b'], (1, 1))
    return wd, ws, w3, w2, w3m, bx, bv2


def _phi_h_weights(g, ln, in_ch):
    ph = g['phi_h']
    wa = ph[0]['w'][:in_ch]
    wb = ph[0]['w'][in_ch:in_ch + 64]
    wn = ph[0]['w'][in_ch + 64]
    vec = jnp.stack([wn, ph[0]['b'], ph[1]['b'], ln['scale'], ln['bias']])
    wh2 = ph[1]['w']
    return wa, wb, vec, wh2


def kernel(x, edge_index, batch, params):
    src = edge_index[0]
    dst = edge_index[1]
    pad = jnp.zeros((EP - E,), jnp.int32)
    dst_p = jnp.concatenate([dst, pad])
    src_p = jnp.concatenate([src, pad])
    zeros_h = jnp.zeros((NP, MW), jnp.float32)
    pv_h = jnp.reshape(x[:, 0:4], (4 * N,))

    g1 = params['g1']
    g2 = params['g2']
    wd1, ws1, w3_1, w2_1, w3m_1, bx_1, bv2_1 = _layer_weights(g1, 128)
    wd2, ws2, w3_2, w2_2, w3m_2, bx_2, bv2_2 = _layer_weights(g2, 64)
    wa1, wb1, vec1, wh21 = _phi_h_weights(g1, params['ln1'], 128)
    wa2, wb2, vec2, wh22 = _phi_h_weights(g2, params['ln2'], 64)

    td, ts = _node_proj(x, wd1, ws1, 128)
    gd, gs, rpx, rpy, dsq, dvr = _gather1_call(td, ts, dst_p, src_p, pv_h)
    m1 = _edge_mlp1(gd, gs, rpx, rpy, dsq, dvr, w3_1, w2_1, w3m_1, bx_1, bv2_1)
    parts1 = _scatter_call(m1, dst_p, zeros_h)

    h, t2d, t2s = _node_mid(parts1, x, wa1, wb1, vec1, wh21, wd2, ws2)

    gd2, gs2 = _gather2_call(t2d, t2s, dst_p, src_p)
    m2 = _edge_mlp2(gd2, gs2, m1, w3_2, w2_2, w3m_2, bx_2, bv2_2)
    parts2 = _scatter_call(m2, dst_p, zeros_h)

    pool = params['pool']
    o0, o1 = params['out'][0], params['out'][1]
    s, latent, mu_c, losses = _final(
        parts2, x, h, wa2, wb2, vec2, wh22,
        pool['w'], jnp.reshape(pool['b'], (1, 32)),
        o0['w'], jnp.reshape(o0['b'], (1, 64)),
        o1['w'], jnp.reshape(o1['b'], (1, 32)))

    return latent, s, losses[0], mu_c


# speedup vs baseline: 1.1057x; 1.0778x over previous
import functools
import jax
import jax.numpy as jnp
from jax import lax
from jax.experimental import pallas as pl
from jax.experimental.pallas import tpu as pltpu
from jax.experimental.pallas import tpu_sc as plsc

N = 10000
E = 320000
NC, NS = 2, 16
NW = NC * NS
EW = 10240
EP = NW * EW
CH = 128
NCHUNK = EW // CH
MW = 128
NP = 10240
NSL = NP // NS
GG = 8
NGB = EP // (CH * GG)


@functools.cache
def _mesh():
    return plsc.VectorSubcoreMesh(
        core_axis_name="c", subcore_axis_name="s", num_cores=NC, num_subcores=NS)


def _silu(v):
    return v * (1.0 / (1.0 + jnp.exp(-v)))


K0 = 80
K1 = 160 - K0


def _gather_base(cid, sid):
    kc = K0 + cid * (K1 - K0)
    base = cid * (NS * K0 * CH) + sid * kc * CH
    return base, kc


@functools.cache
def _make_gather1():
    @functools.partial(
        pl.kernel,
        out_type=(jax.ShapeDtypeStruct((EP, 128), jnp.float32),
                  jax.ShapeDtypeStruct((NGB, GG, 128), jnp.float32),
                  jax.ShapeDtypeStruct((NGB, GG, 128), jnp.float32),
                  jax.ShapeDtypeStruct((NGB, GG, 128), jnp.float32),
                  jax.ShapeDtypeStruct((NGB, GG, 128), jnp.float32)),
        mesh=_mesh(),
        compiler_params=pltpu.CompilerParams(needs_layout_passes=False),
        scratch_types=([pltpu.VMEM((CH,), jnp.int32)] * 4
                       + [pltpu.VMEM((CH, 128), jnp.float32)] * 4
                       + [pltpu.VMEM((4 * N,), jnp.float32),
                          pltpu.VMEM((GG, 128), jnp.float32),
                          pltpu.VMEM((GG, 128), jnp.float32),
                          pltpu.VMEM((GG, 128), jnp.float32),
                          pltpu.VMEM((GG, 128), jnp.float32)]
                       + [pltpu.SemaphoreType.DMA] * 6),
    )
    def gather1_k(td, ts, dsti, srci, pv_h,
                  g, orpx, orpy, odsq, odvr, *sc):
        idxd = sc[0:2]
        idxs = sc[2:4]
        rowd = sc[4:6]
        rowsv = sc[6:8]
        pv, brpx, brpy, bdsq, bdvr = sc[8:13]
        semi = sc[13:15]
        semg = sc[15:17]
        semw = sc[17:19]
        cid = lax.axis_index("c")
        sid = lax.axis_index("s")
        base, kc = _gather_base(cid, sid)
        pltpu.sync_copy(pv_h, pv)

        for b in range(NBUF):
            offp = pl.multiple_of(base + b * CH, CH)
            pltpu.async_copy(dsti.at[pl.ds(offp, CH)], idxd[b], semi[b])
            pltpu.async_copy(srci.at[pl.ds(offp, CH)], idxs[b], semi[b])

        def addrows(p):
            def addbody(r, carry):
                for v in range(128 // 16):
                    sl = pl.ds(v * 16, 16)
                    rowd[p][r, sl] = rowd[p][r, sl] + rowsv[p][r, sl]
                return carry
            lax.fori_loop(0, CH, addbody, 0)

        def body(jo, carry):
            for ji in range(GG):
                b = ji % NBUF
                p = 1 - b
                j = jo * GG + ji
                off = pl.multiple_of(base + j * CH, CH)
                pltpu.make_async_copy(dsti.at[pl.ds(off, CH)], idxd[b], semi[b]).wait()
                pltpu.make_async_copy(srci.at[pl.ds(off, CH)], idxs[b], semi[b]).wait()

                def _drain():
                    pltpu.make_async_copy(rowd[b], g.at[pl.ds(off, CH)], semw[b]).wait()

                if ji < NBUF:
                    pl.when(jo > 0)(_drain)
                else:
                    _drain()

                pltpu.async_copy(td.at[idxd[b]], rowd[b], semg[b])
                pltpu.async_copy(ts.at[idxs[b]], rowsv[b], semg[b])
                for v in range(CH // 16):
                    sl = pl.ds(v * 16, 16)
                    idv = idxd[b][sl] * 4
                    isv = idxs[b][sl] * 4
                    pxd = plsc.load_gather(pv, [idv])
                    pyd = plsc.load_gather(pv, [idv + 1])
                    vxd = plsc.load_gather(pv, [idv + 2])
                    vyd = plsc.load_gather(pv, [idv + 3])
                    pxs = plsc.load_gather(pv, [isv])
                    pys = plsc.load_gather(pv, [isv + 1])
                    vxs = plsc.load_gather(pv, [isv + 2])
                    vys = plsc.load_gather(pv, [isv + 3])
                    rpx = pxs - pxd
                    rpy = pys - pyd
                    rvx = vxs - vxd
                    rvy = vys - vyd
                    brpx[ji, sl] = rpx
                    brpy[ji, sl] = rpy
                    bdsq[ji, sl] = rpx * rpx + rpy * rpy
                    bdvr[ji, sl] = rvx * rpx + rvy * rpy

                def _proc():
                    offm = pl.multiple_of(off - CH, CH)
                    pltpu.make_async_copy(td.at[idxd[p]], rowd[p], semg[p]).wait()
                    pltpu.make_async_copy(ts.at[idxs[p]], rowsv[p], semg[p]).wait()

                    @pl.when(j + 1 < kc)
                    def _prefetch():
                        offn = pl.multiple_of(off + CH, CH)
                        pltpu.async_copy(dsti.at[pl.ds(offn, CH)], idxd[p], semi[p])
                        pltpu.async_copy(srci.at[pl.ds(offn, CH)], idxs[p], semi[p])

                    addrows(p)
                    pltpu.async_copy(rowd[p], g.at[pl.ds(offm, CH)], semw[p])

                if ji == 0:
                    pl.when(jo > 0)(_proc)
                else:
                    _proc()
            gblk = base // (CH * GG) + jo
            pltpu.sync_copy(brpx, orpx.at[gblk])
            pltpu.sync_copy(brpy, orpy.at[gblk])
            pltpu.sync_copy(bdsq, odsq.at[gblk])
            pltpu.sync_copy(bdvr, odvr.at[gblk])
            return carry

        lax.fori_loop(0, kc // GG, body, 0)
        offl = pl.multiple_of(base + (kc - 1) * CH, CH)
        pltpu.make_async_copy(td.at[idxd[1]], rowd[1], semg[1]).wait()
        pltpu.make_async_copy(ts.at[idxs[1]], rowsv[1], semg[1]).wait()
        addrows(1)
        pltpu.async_copy(rowd[1], g.at[pl.ds(offl, CH)], semw[1])
        for b in range(NBUF):
            pltpu.make_async_copy(rowd[b], g.at[pl.ds(offl, CH)], semw[b]).wait()

    return gather1_k


NBUF = 2


@functools.cache
def _make_gather2():
    @functools.partial(
        pl.kernel,
        out_type=jax.ShapeDtypeStruct((EP, 128), jnp.float32),
        mesh=_mesh(),
        compiler_params=pltpu.CompilerParams(needs_layout_passes=False),
        scratch_types=([pltpu.VMEM((CH,), jnp.int32)] * (2 * NBUF)
                       + [pltpu.VMEM((CH, 128), jnp.float32)] * (2 * NBUF)
                       + [pltpu.SemaphoreType.DMA] * (3 * NBUF)),
    )
    def gather2_k(td, ts, dsti, srci, g, *sc):
        idxd = sc[0:NBUF]
        idxs = sc[NBUF:2 * NBUF]
        rowd = sc[2 * NBUF:3 * NBUF]
        rowsv = sc[3 * NBUF:4 * NBUF]
        semi = sc[4 * NBUF:5 * NBUF]
        semg = sc[5 * NBUF:6 * NBUF]
        semw = sc[6 * NBUF:7 * NBUF]
        cid = lax.axis_index("c")
        sid = lax.axis_index("s")
        base, kc = _gather_base(cid, sid)

        for b in range(NBUF):
            offp = pl.multiple_of(base + b * CH, CH)
            pltpu.async_copy(dsti.at[pl.ds(offp, CH)], idxd[b], semi[b])
            pltpu.async_copy(srci.at[pl.ds(offp, CH)], idxs[b], semi[b])

        def addrows(p):
            def addbody(r, carry):
                for v in range(128 // 16):
                    sl = pl.ds(v * 16, 16)
                    rowd[p][r, sl] = rowd[p][r, sl] + rowsv[p][r, sl]
                return carry
            lax.fori_loop(0, CH, addbody, 0)

        def body(jo, carry):
            for ji in range(GG):
                b = ji % NBUF
                p = 1 - b
                j = jo * GG + ji
                off = pl.multiple_of(base + j * CH, CH)
                pltpu.make_async_copy(dsti.at[pl.ds(off, CH)], idxd[b], semi[b]).wait()
                pltpu.make_async_copy(srci.at[pl.ds(off, CH)], idxs[b], semi[b]).wait()

                def _drain():
                    pltpu.make_async_copy(rowd[b], g.at[pl.ds(off, CH)], semw[b]).wait()

                if ji < NBUF:
                    pl.when(jo > 0)(_drain)
                else:
                    _drain()

                pltpu.async_copy(td.at[idxd[b]], rowd[b], semg[b])
                pltpu.async_copy(ts.at[idxs[b]], rowsv[b], semg[b])

                def _proc():
                    offm = pl.multiple_of(off - CH, CH)
                    pltpu.make_async_copy(td.at[idxd[p]], rowd[p], semg[p]).wait()
                    pltpu.make_async_copy(ts.at[idxs[p]], rowsv[p], semg[p]).wait()

                    @pl.when(j + 1 < kc)
                    def _prefetch():
                        offn = pl.multiple_of(off + CH, CH)
                        pltpu.async_copy(dsti.at[pl.ds(offn, CH)], idxd[p], semi[p])
                        pltpu.async_copy(srci.at[pl.ds(offn, CH)], idxs[p], semi[p])

                    addrows(p)
                    pltpu.async_copy(rowd[p], g.at[pl.ds(offm, CH)], semw[p])

                if ji == 0:
                    pl.when(jo > 0)(_proc)
                else:
                    _proc()
            return carry

        lax.fori_loop(0, kc // GG, body, 0)
        offl = pl.multiple_of(base + (kc - 1) * CH, CH)
        pltpu.make_async_copy(td.at[idxd[1]], rowd[1], semg[1]).wait()
        pltpu.make_async_copy(ts.at[idxs[1]], rowsv[1], semg[1]).wait()
        addrows(1)
        pltpu.async_copy(rowd[1], g.at[pl.ds(offl, CH)], semw[1])
        for b in range(NBUF):
            pltpu.make_async_copy(rowd[b], g.at[pl.ds(offl, CH)], semw[b]).wait()

    return gather2_k


@functools.cache
def _make_scatter():
    @functools.partial(
        pl.kernel,
        out_type=jax.ShapeDtypeStruct((NC * NP, MW), jnp.float32),
        mesh=_mesh(),
        compiler_params=pltpu.CompilerParams(needs_layout_passes=False),
        scratch_types=[pltpu.VMEM((CH,), jnp.int32),
                       pltpu.VMEM((CH, MW), jnp.float32),
                       pltpu.VMEM_SHARED((NP, MW), jnp.float32)],
    )
    def scatter_k(m, dsti, zeros_h, out, idxv, rowv, acc):
        cid = lax.axis_index("c")
        sid = lax.axis_index("s")
        pltpu.sync_copy(zeros_h.at[pl.ds(sid * NSL, NSL)],
                        acc.at[pl.ds(sid * NSL, NSL)])
        plsc.subcore_barrier()
        base = (sid * NC + cid) * EW

        def body(j, carry):
            off = pl.multiple_of(base + j * CH, CH)
            pltpu.sync_copy(dsti.at[pl.ds(off, CH)], idxv)
            pltpu.sync_copy(m.at[pl.ds(off, CH)], rowv)
            pltpu.sync_copy(rowv, acc.at[idxv], add=True)
            return carry

        lax.fori_loop(0, NCHUNK, body, 0)
        plsc.subcore_barrier()
        pltpu.sync_copy(acc.at[pl.ds(sid * NSL, NSL)],
                        out.at[pl.ds(cid * NP + sid * NSL, NSL)])

    return scatter_k


def _gather1_call(td, ts, dsti, srci, pv_h):
    return _make_gather1()(td, ts, dsti, srci, pv_h)


def _gather2_call(td, ts, dsti, srci):
    return _make_gather2()(td, ts, dsti, srci)


def _scatter_call(m, dsti, zeros_h):
    out = _make_scatter()(m, dsti, zeros_h)
    return jnp.reshape(out, (NC, NP, MW))[:, :N, :]


BLK_N = 1000
BLK_E = 1024
GB = BLK_E // CH


def _node_proj(x, wd, ws, cin):
    def body(x_ref, wd_ref, ws_ref, td_ref, ts_ref):
        xb = x_ref[...]
        td_ref[...] = jnp.dot(xb, wd_ref[...], preferred_element_type=jnp.float32)
        ts_ref[...] = jnp.dot(xb, ws_ref[...], preferred_element_type=jnp.float32)

    return pl.pallas_call(
        body,
        grid=(N // BLK_N,),
        in_specs=[pl.BlockSpec((BLK_N, cin), lambda i: (i, 0)),
                  pl.BlockSpec((cin, 128), lambda i: (0, 0)),
                  pl.BlockSpec((cin, 128), lambda i: (0, 0))],
        out_specs=[pl.BlockSpec((BLK_N, 128), lambda i: (i, 0)),
                   pl.BlockSpec((BLK_N, 128), lambda i: (i, 0))],
        out_shape=[jax.ShapeDtypeStruct((N, 128), jnp.float32)] * 2,
    )(x, wd, ws)


def _edge_core(gb, rp, dsq, dvr, w3_ref, w2_ref, w3m_ref, bx_ref, bv2_ref):
    pre = (gb + w3_ref[0:1, :]
           + dsq * w3_ref[1:2, :] + dvr * w3_ref[2:3, :])
    h1e = _silu(pre[:, 0:64])
    h1v = _silu(pre[:, 64:128])
    h2 = _silu(jnp.dot(h1e, w2_ref[...], preferred_element_type=jnp.float32)
               + bx_ref[0:1, :])
    mh = jnp.dot(h2, w3m_ref[...], preferred_element_type=jnp.float32) + bx_ref[1:2, :]
    vw = jnp.sum(h1v * bx_ref[2:3, :], axis=1, keepdims=True) + bv2_ref[0:1, 0:1]
    mv = vw * rp
    rowid = (jax.lax.broadcasted_iota(jnp.int32, (BLK_E, 1), 0)
             + pl.program_id(0) * BLK_E)
    mrow = jnp.concatenate(
        [mh, mv, rp, dsq, dvr, jnp.zeros((BLK_E, MW - 70), jnp.float32)], axis=1)
    return jnp.where(rowid < E, mrow, 0.0)


def _edge_mlp1(g, rpx, rpy, dsq, dvr, w3, w2, w3m, bx, bv2):
    def body(g_ref, rpx_ref, rpy_ref, dsq_ref, dvr_ref,
             w3_ref, w2_ref, w3m_ref, bx_ref, bv2_ref, m_ref):
        r = jax.lax.broadcasted_iota(jnp.int32, (BLK_E, 1), 0)
        sub = jax.lax.broadcasted_iota(jnp.int32, (BLK_E, GG), 1)
        iexp = jnp.where(sub == (r >> 7), 1.0, 0.0)
        lane = jax.lax.broadcasted_iota(jnp.int32, (BLK_E, 128), 1)
        msel = jnp.where(lane == (r & 127), 1.0, 0.0)

        def expand(p_ref):
            c = jnp.dot(iexp, p_ref[0], preferred_element_type=jnp.float32)
            return jnp.sum(c * msel, axis=1, keepdims=True)

        rp = jnp.concatenate([expand(rpx_ref), expand(rpy_ref)], axis=1)
        dsqc = expand(dsq_ref)
        dvrc = expand(dvr_ref)
        m_ref[...] = _edge_core(g_ref[...], rp, dsqc, dvrc,
                                w3_ref, w2_ref, w3m_ref, bx_ref, bv2_ref)

    return pl.pallas_call(
        body,
        grid=(EP // BLK_E,),
        in_specs=[pl.BlockSpec((BLK_E, 128), lambda i: (i, 0)),
                  pl.BlockSpec((1, GG, 128), lambda i: (i, 0, 0)),
                  pl.BlockSpec((1, GG, 128), lambda i: (i, 0, 0)),
                  pl.BlockSpec((1, GG, 128), lambda i: (i, 0, 0)),
                  pl.BlockSpec((1, GG, 128), lambda i: (i, 0, 0)),
                  pl.BlockSpec((3, 128), lambda i: (0, 0)),
                  pl.BlockSpec((64, 64), lambda i: (0, 0)),
                  pl.BlockSpec((64, 64), lambda i: (0, 0)),
                  pl.BlockSpec((3, 64), lambda i: (0, 0)),
                  pl.BlockSpec((1, 1), lambda i: (0, 0))],
        out_specs=pl.BlockSpec((BLK_E, MW), lambda i: (i, 0)),
        out_shape=jax.ShapeDtypeStruct((EP, MW), jnp.float32),
    )(g, rpx, rpy, dsq, dvr, w3, w2, w3m, bx, bv2)


def _edge_mlp2(g, m1, w3, w2, w3m, bx, bv2):
    def body(g_ref, m1_ref, w3_ref, w2_ref, w3m_ref, bx_ref, bv2_ref,
             m_ref):
        geo = m1_ref[:, 64:80]
        rp = geo[:, 2:4]
        dsq = geo[:, 4:5]
        dvr = geo[:, 5:6]
        m_ref[...] = _edge_core(g_ref[...], rp, dsq, dvr,
                                w3_ref, w2_ref, w3m_ref, bx_ref, bv2_ref)

    return pl.pallas_call(
        body,
        grid=(EP // BLK_E,),
        in_specs=[pl.BlockSpec((BLK_E, 128), lambda i: (i, 0)),
                  pl.BlockSpec((BLK_E, MW), lambda i: (i, 0)),
                  pl.BlockSpec((3, 128), lambda i: (0, 0)),
                  pl.BlockSpec((64, 64), lambda i: (0, 0)),
                  pl.BlockSpec((64, 64), lambda i: (0, 0)),
                  pl.BlockSpec((3, 64), lambda i: (0, 0)),
                  pl.BlockSpec((1, 1), lambda i: (0, 0))],
        out_specs=pl.BlockSpec((BLK_E, MW), lambda i: (i, 0)),
        out_shape=jax.ShapeDtypeStruct((EP, MW), jnp.float32),
    )(g, m1, w3, w2, w3m, bx, bv2)


def _node_mid(parts, x, wa, wb, vec, wh2, w2d, w2s):
    def body(p_ref, x_ref, wa_ref, wb_ref, vec_ref, wh2_ref, w2d_ref, w2s_ref,
             h_ref, t2d_ref, t2s_ref):
        p = p_ref[0] + p_ref[1]
        mh = p[:, 0:64]
        mv = p[:, 64:66]
        nrm = jnp.sqrt(jnp.sum(mv * mv, axis=1, keepdims=True) + 1e-12)
        pre = (jnp.dot(x_ref[...], wa_ref[...], preferred_element_type=jnp.float32)
               + jnp.dot(mh, wb_ref[...], preferred_element_type=jnp.float32)
               + nrm * vec_ref[0:1, :] + vec_ref[1:2, :])
        hh = jnp.dot(_silu(pre), wh2_ref[...], preferred_element_type=jnp.float32) + vec_ref[2:3, :]
        g = jnp.maximum(hh, 0.0)
        mu = jnp.mean(g, axis=1, keepdims=True)
        var = jnp.mean(g * g, axis=1, keepdims=True) - mu * mu
        hb = (g - mu) * jax.lax.rsqrt(var + 1e-5) * vec_ref[3:4, :] + vec_ref[4:5, :]
        h_ref[...] = hb
        t2d_ref[...] = jnp.dot(hb, w2d_ref[...], preferred_element_type=jnp.float32)
        t2s_ref[...] = jnp.dot(hb, w2s_ref[...], preferred_element_type=jnp.float32)

    return pl.pallas_call(
        body,
        grid=(N // BLK_N,),
        in_specs=[pl.BlockSpec((2, BLK_N, MW), lambda i: (0, i, 0)),
                  pl.BlockSpec((BLK_N, 128), lambda i: (i, 0)),
                  pl.BlockSpec((128, 64), lambda i: (0, 0)),
                  pl.BlockSpec((64, 64), lambda i: (0, 0)),
                  pl.BlockSpec((5, 64), lambda i: (0, 0)),
                  pl.BlockSpec((64, 64), lambda i: (0, 0)),
                  pl.BlockSpec((64, 128), lambda i: (0, 0)),
                  pl.BlockSpec((64, 128), lambda i: (0, 0))],
        out_specs=[pl.BlockSpec((BLK_N, 64), lambda i: (i, 0)),
                   pl.BlockSpec((BLK_N, 128), lambda i: (i, 0)),
                   pl.BlockSpec((BLK_N, 128), lambda i: (i, 0))],
        out_shape=[jax.ShapeDtypeStruct((N, 64), jnp.float32),
                   jax.ShapeDtypeStruct((N, 128), jnp.float32),
                   jax.ShapeDtypeStruct((N, 128), jnp.float32)],
    )(parts, x, wa, wb, vec, wh2, w2d, w2s)


def _final(parts, x, h, wa, wb, vec, wh2, poolw, poolb, o0, o0b, o1, o1b):
    nblk = N // BLK_N

    def body(p_ref, x_ref, h_ref, wa_ref, wb_ref, vec_ref, wh2_ref,
             poolw_ref, poolb_ref, o0_ref, o0b_ref, o1_ref, o1b_ref,
             s_ref, lat_ref, mu_ref, loss_ref,
             acc_cs, acc_sth, acc_stp, acc_ent):
        i = pl.program_id(0)

        @pl.when(i == 0)
        def _init():
            acc_cs[...] = jnp.zeros_like(acc_cs)
            acc_sth[...] = jnp.zeros_like(acc_sth)
            acc_stp[...] = jnp.zeros_like(acc_stp)
            acc_ent[...] = jnp.zeros_like(acc_ent)

        p = p_ref[0] + p_ref[1]
        mh = p[:, 0:64]
        mv = p[:, 64:66]
        nrm = jnp.sqrt(jnp.sum(mv * mv, axis=1, keepdims=True) + 1e-12)
        pre = (jnp.dot(h_ref[...], wa_ref[...], preferred_element_type=jnp.float32)
               + jnp.dot(mh, wb_ref[...], preferred_element_type=jnp.float32)
               + nrm * vec_ref[0:1, :] + vec_ref[1:2, :])
        hh = jnp.dot(_silu(pre), wh2_ref[...], preferred_element_type=jnp.float32) + vec_ref[2:3, :]
        g = jnp.maximum(hh, 0.0)
        mu = jnp.mean(g, axis=1, keepdims=True)
        var = jnp.mean(g * g, axis=1, keepdims=True) - mu * mu
        h2 = (g - mu) * jax.lax.rsqrt(var + 1e-5) * vec_ref[3:4, :] + vec_ref[4:5, :]

        logits = jnp.dot(h2, poolw_ref[...], preferred_element_type=jnp.float32) + poolb_ref[...]
        mx = jnp.max(logits, axis=1, keepdims=True)
        ex = jnp.exp(logits - mx)
        sm = ex / jnp.sum(ex, axis=1, keepdims=True)
        s_ref[...] = sm

        dn = (((0,), (0,)), ((), ()))
        ones = jnp.ones((BLK_N, 1), jnp.float32)
        acc_cs[...] += jax.lax.dot_general(sm, ones, dn, preferred_element_type=jnp.float32)
        acc_sth[...] += jax.lax.dot_general(sm, h2, dn, preferred_element_type=jnp.float32)
        acc_stp[...] += jax.lax.dot_general(sm, x_ref[:, 0:2], dn, preferred_element_type=jnp.float32)
        acc_ent[...] += jnp.reshape(jnp.sum(sm * jnp.log(sm + 1e-8)), (1, 1))

        @pl.when(i == nblk - 1)
        def _fin():
            cs = acc_cs[...]
            denom = cs + 1e-8
            pooled = acc_sth[...] / denom
            mu_ref[...] = acc_stp[...] / denom
            lat = jnp.dot(
                jnp.maximum(jnp.dot(pooled, o0_ref[...], preferred_element_type=jnp.float32)
                            + o0b_ref[...], 0.0),
                o1_ref[...], preferred_element_type=jnp.float32) + o1b_ref[...]
            lat_ref[...] = lat
            mean_s = cs * (1.0 / N)
            load = jnp.sum((mean_s - 1.0 / 32.0) ** 2)
            loss_ref[...] = jnp.concatenate(
                [acc_ent[...] * (-1.0 / N), jnp.reshape(load, (1, 1))], axis=1)

    return pl.pallas_call(
        body,
        grid=(nblk,),
        in_specs=[pl.BlockSpec((2, BLK_N, MW), lambda i: (0, i, 0)),
                  pl.BlockSpec((BLK_N, 128), lambda i: (i, 0)),
                  pl.BlockSpec((BLK_N, 64), lambda i: (i, 0)),
                  pl.BlockSpec((64, 64), lambda i: (0, 0)),
                  pl.BlockSpec((64, 64), lambda i: (0, 0)),
                  pl.BlockSpec((5, 64), lambda i: (0, 0)),
                  pl.BlockSpec((64, 64), lambda i: (0, 0)),
                  pl.BlockSpec((64, 32), lambda i: (0, 0)),
                  pl.BlockSpec((1, 32), lambda i: (0, 0)),
                  pl.BlockSpec((64, 64), lambda i: (0, 0)),
                  pl.BlockSpec((1, 64), lambda i: (0, 0)),
                  pl.BlockSpec((64, 32), lambda i: (0, 0)),
                  pl.BlockSpec((1, 32), lambda i: (0, 0))],
        out_specs=[pl.BlockSpec((BLK_N, 32), lambda i: (i, 0)),
                   pl.BlockSpec((32, 32), lambda i: (0, 0)),
                   pl.BlockSpec((32, 2), lambda i: (0, 0)),
                   pl.BlockSpec((1, 2), lambda i: (0, 0))],
        out_shape=[jax.ShapeDtypeStruct((N, 32), jnp.float32),
                   jax.ShapeDtypeStruct((32, 32), jnp.float32),
                   jax.ShapeDtypeStruct((32, 2), jnp.float32),
                   jax.ShapeDtypeStruct((1, 2), jnp.float32)],
        scratch_shapes=[pltpu.VMEM((32, 1), jnp.float32),
                        pltpu.VMEM((32, 64), jnp.float32),
                        pltpu.VMEM((32, 2), jnp.float32),
                        pltpu.VMEM((1, 1), jnp.float32)],
    )(parts, x, h, wa, wb, vec, wh2, poolw, poolb, o0, o0b, o1, o1b)


def _layer_weights(g, in_ch):
    we0, wv0 = g['phi_e'][0]['w'], g['phi_v'][0]['w']
    wd = jnp.concatenate([we0[:in_ch], wv0[:in_ch]], axis=1)
    ws = jnp.concatenate([we0[in_ch:2 * in_ch], wv0[in_ch:2 * in_ch]], axis=1)
    b1 = jnp.concatenate([g['phi_e'][0]['b'], g['phi_v'][0]['b']])
    wdsq = jnp.concatenate([we0[2 * in_ch], wv0[2 * in_ch]])
    wdvr = jnp.concatenate([we0[2 * in_ch + 1], wv0[2 * in_ch + 1]])
    w3 = jnp.stack([b1, wdsq, wdvr])
    w2 = g['phi_e'][1]['w']
    w3m = g['phi_e'][2]['w']
    bx = jnp.stack([g['phi_e'][1]['b'], g['phi_e'][2]['b'],
                    g['phi_v'][1]['w'][:, 0]])
    bv2 = jnp.reshape(g['phi_v'][1]['b'], (1, 1))
    return wd, ws, w3, w2, w3m, bx, bv2


def _phi_h_weights(g, ln, in_ch):
    ph = g['phi_h']
    wa = ph[0]['w'][:in_ch]
    wb = ph[0]['w'][in_ch:in_ch + 64]
    wn = ph[0]['w'][in_ch + 64]
    vec = jnp.stack([wn, ph[0]['b'], ph[1]['b'], ln['scale'], ln['bias']])
    wh2 = ph[1]['w']
    return wa, wb, vec, wh2


def kernel(x, edge_index, batch, params):
    src = edge_index[0]
    dst = edge_index[1]
    pad = jnp.zeros((EP - E,), jnp.int32)
    dst_p = jnp.concatenate([dst, pad])
    src_p = jnp.concatenate([src, pad])
    zeros_h = jnp.zeros((NP, MW), jnp.float32)
    pv_h = jnp.reshape(x[:, 0:4], (4 * N,))

    g1 = params['g1']
    g2 = params['g2']
    wd1, ws1, w3_1, w2_1, w3m_1, bx_1, bv2_1 = _layer_weights(g1, 128)
    wd2, ws2, w3_2, w2_2, w3m_2, bx_2, bv2_2 = _layer_weights(g2, 64)
    wa1, wb1, vec1, wh21 = _phi_h_weights(g1, params['ln1'], 128)
    wa2, wb2, vec2, wh22 = _phi_h_weights(g2, params['ln2'], 64)

    td, ts = _node_proj(x, wd1, ws1, 128)
    g1k, rpx, rpy, dsq, dvr = _gather1_call(td, ts, dst_p, src_p, pv_h)
    m1 = _edge_mlp1(g1k, rpx, rpy, dsq, dvr, w3_1, w2_1, w3m_1, bx_1, bv2_1)
    parts1 = _scatter_call(m1, dst_p, zeros_h)

    h, t2d, t2s = _node_mid(parts1, x, wa1, wb1, vec1, wh21, wd2, ws2)

    g2k = _gather2_call(t2d, t2s, dst_p, src_p)
    m2 = _edge_mlp2(g2k, m1, w3_2, w2_2, w3m_2, bx_2, bv2_2)
    parts2 = _scatter_call(m2, dst_p, zeros_h)

    pool = params['pool']
    o0, o1 = params['out'][0], params['out'][1]
    s, latent, mu_c, losses = _final(
        parts2, x, h, wa2, wb2, vec2, wh22,
        pool['w'], jnp.reshape(pool['b'], (1, 32)),
        o0['w'], jnp.reshape(o0['b'], (1, 64)),
        o1['w'], jnp.reshape(o1['b'], (1, 32)))

    return latent, s, losses[0], mu_c
